# baseline (speedup 1.0000x reference)
import jax
import jax.numpy as jnp
from jax import lax
from jax.experimental import pallas as pl
from jax.experimental.pallas import tpu as pltpu
from jax.experimental.pallas import tpu_sc as plsc

_SPACING = (0.001, 0.001)
_N_KEYS = 2048
_N_QUERIES = 3 * _N_KEYS
_D = 128
_LANES = 16
_NUM_CORES = 2
_NUM_SUBCORES = 16
_NUM_WORKERS = _NUM_CORES * _NUM_SUBCORES
_QPW = _N_QUERIES // _NUM_WORKERS
_GROUPS = _QPW // _LANES


def _sc_body(kx_hbm, ky_hbm, qx_hbm, qy_hbm, values_hbm, out_hbm,
             kx_v, ky_v, qx_v, qy_v, rows_v, sem):
    wid = lax.axis_index("s") * _NUM_CORES + lax.axis_index("c")
    base = wid * _QPW

    pltpu.sync_copy(kx_hbm, kx_v)
    pltpu.sync_copy(ky_hbm, ky_v)
    pltpu.sync_copy(qx_hbm.at[pl.ds(base, _QPW)], qx_v)
    pltpu.sync_copy(qy_hbm.at[pl.ds(base, _QPW)], qy_v)

    copies = []
    for g in range(_GROUPS):
        qxv = qx_v[pl.ds(g * _LANES, _LANES)]
        qyv = qy_v[pl.ds(g * _LANES, _LANES)]
        init = (
            jnp.full((_LANES,), jnp.inf, jnp.float32),
            jnp.zeros((_LANES,), jnp.int32),
            jnp.zeros((_LANES,), jnp.int32),
        )

        def chunk_body(c, carry):
            best_d, best_i, kvec = carry
            kxc = kx_v[pl.ds(c * _LANES, _LANES)]
            kyc = ky_v[pl.ds(c * _LANES, _LANES)]
            for j in range(_LANES):
                dx = qxv - kxc[j]
                dy = qyv - kyc[j]
                d = dx * dx + dy * dy
                pred = d < best_d
                best_d = jnp.where(pred, d, best_d)
                best_i = jnp.where(pred, kvec, best_i)
                kvec = kvec + 1
            return best_d, best_i, kvec

        _, best_i, _ = lax.fori_loop(0, _N_KEYS // _LANES, chunk_body, init)
        copies.append(
            pltpu.async_copy(
                values_hbm.at[best_i],
                rows_v.at[pl.ds(g * _LANES, _LANES)],
                sem,
            )
        )
    for c in copies:
        c.wait()
    pltpu.sync_copy(rows_v, out_hbm.at[pl.ds(base, _QPW)])


def _nn_gather(kx, ky, qx, qy, values_t):
    mesh = plsc.VectorSubcoreMesh(core_axis_name="c", subcore_axis_name="s")
    return pl.kernel(
        _sc_body,
        out_type=jax.ShapeDtypeStruct((_N_QUERIES, _D), jnp.float32),
        mesh=mesh,
        scratch_types=[
            pltpu.VMEM((_N_KEYS,), jnp.float32),
            pltpu.VMEM((_N_KEYS,), jnp.float32),
            pltpu.VMEM((_QPW,), jnp.float32),
            pltpu.VMEM((_QPW,), jnp.float32),
            pltpu.VMEM((_QPW, _D), jnp.float32),
            pltpu.SemaphoreType.DMA,
        ],
    )(kx, ky, qx, qy, values_t)


def kernel(values, coords):
    sx, sy = _SPACING
    shift = jnp.asarray([sx / 2.0, sy / 2.0], dtype=values.dtype)
    x = coords[:, 0]
    y = coords[:, 1]
    new_coords = jnp.concatenate(
        (
            jnp.stack((x, y + sy), axis=1),
            jnp.stack((x + sx, y), axis=1),
            jnp.stack((x + sx, y + sy), axis=1),
        ),
        axis=0,
    )
    sampled_coords = jnp.concatenate((coords, new_coords), axis=0)
    q = new_coords - shift
    out_t = _nn_gather(x, y, q[:, 0], q[:, 1], values.T)
    out_values = jnp.concatenate((values, out_t.T), axis=1)
    return out_values, sampled_coords


# speedup vs baseline: 1.2907x; 1.2907x over previous
import jax
import jax.numpy as jnp
from jax import lax
from jax.experimental import pallas as pl
from jax.experimental.pallas import tpu as pltpu
from jax.experimental.pallas import tpu_sc as plsc

_SPACING = (0.001, 0.001)
_N_KEYS = 2048
_N_QUERIES = 3 * _N_KEYS
_D = 128
_LANES = 16
_NUM_CORES = 2
_NUM_SUBCORES = 16
_NUM_WORKERS = _NUM_CORES * _NUM_SUBCORES
_QPW = _N_QUERIES // _NUM_WORKERS
_GROUPS = _QPW // _LANES


def _walk(skx_v, sky_v, sidx_v, qxv, qyv, p0, bd, bi, going_right):
    delta = 1 if going_right else -1

    def cond(state):
        return jnp.any(state[3])

    def body(state):
        p, bd, bi, active = state
        pc = jnp.clip(p, 0, _N_KEYS - 1)
        kxp = plsc.load_gather(skx_v, [pc])
        kyp = plsc.load_gather(sky_v, [pc])
        oip = plsc.load_gather(sidx_v, [pc])
        dx = qxv - kxp
        dy = qyv - kyp
        dxx = dx * dx
        d = dxx + dy * dy
        inb = (p >= 0) & (p < _N_KEYS)
        valid = active & inb
        better = (d < bd) | ((d == bd) & (oip < bi))
        take = valid & better
        bd = jnp.where(take, d, bd)
        bi = jnp.where(take, oip, bi)
        not_past = (dx > 0.0) if going_right else (dx < 0.0)
        nactive = valid & (not_past | (dxx <= bd))
        return p + delta, bd, bi, nactive

    _, bd, bi, _ = lax.while_loop(
        cond, body, (p0, bd, bi, jnp.full((_LANES,), True))
    )
    return bd, bi


def _sc_body(skx_hbm, sky_hbm, sidx_hbm, inv_hbm, qx_hbm, qy_hbm, values_hbm,
             out_hbm, skx_v, sky_v, sidx_v, inv_v, qx_v, qy_v, rows_v, sem):
    wid = lax.axis_index("s") * _NUM_CORES + lax.axis_index("c")
    base = wid * _QPW

    pltpu.sync_copy(skx_hbm, skx_v)
    pltpu.sync_copy(sky_hbm, sky_v)
    pltpu.sync_copy(sidx_hbm, sidx_v)
    pltpu.sync_copy(inv_hbm, inv_v)
    pltpu.sync_copy(qx_hbm.at[pl.ds(base, _QPW)], qx_v)
    pltpu.sync_copy(qy_hbm.at[pl.ds(base, _QPW)], qy_v)

    copies = []
    for g in range(_GROUPS):
        qxv = qx_v[pl.ds(g * _LANES, _LANES)]
        qyv = qy_v[pl.ds(g * _LANES, _LANES)]
        qidx = base + g * _LANES + lax.iota(jnp.int32, _LANES)
        src = qidx & (_N_KEYS - 1)
        p0 = plsc.load_gather(inv_v, [src])
        bd = jnp.full((_LANES,), jnp.inf, jnp.float32)
        bi = jnp.full((_LANES,), _N_KEYS, jnp.int32)
        bd, bi = _walk(skx_v, sky_v, sidx_v, qxv, qyv, p0, bd, bi, True)
        bd, bi = _walk(skx_v, sky_v, sidx_v, qxv, qyv, p0 - 1, bd, bi, False)
        copies.append(
            pltpu.async_copy(
                values_hbm.at[bi],
                rows_v.at[pl.ds(g * _LANES, _LANES)],
                sem,
            )
        )
    for c in copies:
        c.wait()
    pltpu.sync_copy(rows_v, out_hbm.at[pl.ds(base, _QPW)])


def _nn_gather(skx, sky, sidx, inv, qx, qy, values_t):
    mesh = plsc.VectorSubcoreMesh(core_axis_name="c", subcore_axis_name="s")
    return pl.kernel(
        _sc_body,
        out_type=jax.ShapeDtypeStruct((_N_QUERIES, _D), jnp.float32),
        mesh=mesh,
        compiler_params=pltpu.CompilerParams(needs_layout_passes=False),
        scratch_types=[
            pltpu.VMEM((_N_KEYS,), jnp.float32),
            pltpu.VMEM((_N_KEYS,), jnp.float32),
            pltpu.VMEM((_N_KEYS,), jnp.int32),
            pltpu.VMEM((_N_KEYS,), jnp.int32),
            pltpu.VMEM((_QPW,), jnp.float32),
            pltpu.VMEM((_QPW,), jnp.float32),
            pltpu.VMEM((_QPW, _D), jnp.float32),
            pltpu.SemaphoreType.DMA,
        ],
    )(skx, sky, sidx, inv, qx, qy, values_t)


def kernel(values, coords):
    sx, sy = _SPACING
    shift = jnp.asarray([sx / 2.0, sy / 2.0], dtype=values.dtype)
    x = coords[:, 0]
    y = coords[:, 1]
    new_coords = jnp.concatenate(
        (
            jnp.stack((x, y + sy), axis=1),
            jnp.stack((x + sx, y), axis=1),
            jnp.stack((x + sx, y + sy), axis=1),
        ),
        axis=0,
    )
    sampled_coords = jnp.concatenate((coords, new_coords), axis=0)
    q = new_coords - shift
    order = jnp.argsort(x).astype(jnp.int32)
    skx = x[order]
    sky = y[order]
    inv = jnp.zeros((_N_KEYS,), jnp.int32).at[order].set(
        jnp.arange(_N_KEYS, dtype=jnp.int32)
    )
    out_t = _nn_gather(skx, sky, order, inv, q[:, 0], q[:, 1], values.T)
    out_values = jnp.concatenate((values, out_t.T), axis=1)
    return out_values, sampled_coords


# speedup vs baseline: 1.9114x; 1.4809x over previous
import jax
import jax.numpy as jnp
from jax import lax
from jax.experimental import pallas as pl
from jax.experimental.pallas import tpu as pltpu
from jax.experimental.pallas import tpu_sc as plsc

_SPACING = (0.001, 0.001)
_N_KEYS = 2048
_N_QUERIES = 3 * _N_KEYS
_D = 128
_LANES = 16
_CHUNKS = _N_KEYS // _LANES
_NUM_CORES = 2
_NUM_SUBCORES = 16
_NUM_WORKERS = _NUM_CORES * _NUM_SUBCORES
_QPW = _N_QUERIES // _NUM_WORKERS
_GROUPS = _QPW // _LANES


def _walk(skx_v, sky_v, qxv, qyv, p0, bd, bp, going_right):
    delta = 1 if going_right else -1

    def cond(state):
        return jnp.any(state[3])

    def body(state):
        p, bd, bp, active = state
        pc = jnp.clip(p, 0, _N_KEYS - 1)
        kxp = plsc.load_gather(skx_v, [pc])
        kyp = plsc.load_gather(sky_v, [pc])
        dx = qxv - kxp
        dy = qyv - kyp
        dxx = dx * dx
        d = dxx + dy * dy
        inb = (p >= 0) & (p < _N_KEYS)
        valid = active & inb
        take = valid & (d < bd)
        bd = jnp.where(take, d, bd)
        bp = jnp.where(take, p, bp)
        not_past = (dx > 0.0) if going_right else (dx < 0.0)
        nactive = valid & (not_past | (dxx <= bd))
        return p + delta, bd, bp, nactive

    _, bd, bp, _ = lax.while_loop(
        cond, body, (p0, bd, bp, jnp.full((_LANES,), True))
    )
    return bd, bp


def _sc_body(x_hbm, y_hbm, order_hbm, values_hbm, out_hbm,
             x_v, y_v, order_v, skx_v, sky_v, inv_v, rows_v, sem):
    wid = lax.axis_index("s") * _NUM_CORES + lax.axis_index("c")
    base = wid * _QPW

    pltpu.sync_copy(x_hbm, x_v)
    pltpu.sync_copy(y_hbm, y_v)
    pltpu.sync_copy(order_hbm, order_v)

    lane = lax.iota(jnp.int32, _LANES)

    def sort_body(c, _):
        pos = c * _LANES + lane
        ov = order_v[pl.ds(c * _LANES, _LANES)]
        skx_v[pl.ds(c * _LANES, _LANES)] = plsc.load_gather(x_v, [ov])
        sky_v[pl.ds(c * _LANES, _LANES)] = plsc.load_gather(y_v, [ov])
        plsc.store_scatter(inv_v, [ov], pos)
        return 0

    lax.fori_loop(0, _CHUNKS, sort_body, 0)

    sx, sy = _SPACING
    half_x = jnp.float32(sx / 2.0)
    half_y = jnp.float32(sy / 2.0)

    copies = []
    for g in range(_GROUPS):
        qidx = base + g * _LANES + lane
        src = qidx & (_N_KEYS - 1)
        copy_id = qidx >> 11
        xs = plsc.load_gather(x_v, [src])
        ys = plsc.load_gather(y_v, [src])
        qxv = jnp.where(copy_id == 0, xs - half_x, (xs + sx) - half_x)
        qyv = jnp.where(copy_id == 1, ys - half_y, (ys + sy) - half_y)
        p0 = plsc.load_gather(inv_v, [src])
        bd = jnp.full((_LANES,), jnp.inf, jnp.float32)
        bp = jnp.zeros((_LANES,), jnp.int32)
        bd, bp = _walk(skx_v, sky_v, qxv, qyv, p0, bd, bp, True)
        bd, bp = _walk(skx_v, sky_v, qxv, qyv, p0 - 1, bd, bp, False)
        bi = plsc.load_gather(order_v, [bp])
        copies.append(
            pltpu.async_copy(
                values_hbm.at[bi],
                rows_v.at[pl.ds(g * _LANES, _LANES)],
                sem,
            )
        )
    for c in copies:
        c.wait()
    pltpu.sync_copy(rows_v, out_hbm.at[pl.ds(base, _QPW)])


def _nn_gather(x, y, order, values_t):
    mesh = plsc.VectorSubcoreMesh(core_axis_name="c", subcore_axis_name="s")
    return pl.kernel(
        _sc_body,
        out_type=jax.ShapeDtypeStruct((_N_QUERIES, _D), jnp.float32),
        mesh=mesh,
        compiler_params=pltpu.CompilerParams(needs_layout_passes=False),
        scratch_types=[
            pltpu.VMEM((_N_KEYS,), jnp.float32),
            pltpu.VMEM((_N_KEYS,), jnp.float32),
            pltpu.VMEM((_N_KEYS,), jnp.int32),
            pltpu.VMEM((_N_KEYS,), jnp.float32),
            pltpu.VMEM((_N_KEYS,), jnp.float32),
            pltpu.VMEM((_N_KEYS,), jnp.int32),
            pltpu.VMEM((_QPW, _D), jnp.float32),
            pltpu.SemaphoreType.DMA,
        ],
    )(x, y, order, values_t)


def kernel(values, coords):
    sx, sy = _SPACING
    x = coords[:, 0]
    y = coords[:, 1]
    new_coords = jnp.concatenate(
        (
            jnp.stack((x, y + sy), axis=1),
            jnp.stack((x + sx, y), axis=1),
            jnp.stack((x + sx, y + sy), axis=1),
        ),
        axis=0,
    )
    sampled_coords = jnp.concatenate((coords, new_coords), axis=0)
    order = jnp.argsort(x).astype(jnp.int32)
    out_t = _nn_gather(x, y, order, values.T)
    out_values = jnp.concatenate((values, out_t.T), axis=1)
    return out_values, sampled_coords
